# baseline (speedup 1.0000x reference)
import functools

import jax
import jax.numpy as jnp
from jax import lax
from jax.experimental import pallas as pl
from jax.experimental.pallas import tpu as pltpu
from jax.experimental.pallas import tpu_sc as plsc

_B, _T, _S = 16, 2048, 512
_NROWS = _B * _T
_NC, _NS, _L = 2, 16, 16
_NW = _NC * _NS
_ROWS_PER_W = _NROWS // _NW
_CHUNK = 32
_NCHUNK = _ROWS_PER_W // _CHUNK


def _roll_body(src_hbm, shifts_hbm, out_hbm, shifts_v,
               inb0, inb1, outb0, outb1, si0, si1, so0, so1):
    wid = lax.axis_index("s") * _NC + lax.axis_index("c")
    b = wid // 2
    t0 = (wid % 2) * _ROWS_PER_W
    pltpu.sync_copy(shifts_hbm.at[b, pl.ds(t0, _ROWS_PER_W)], shifts_v)

    iota = lax.iota(jnp.int32, _L)
    zero16 = iota * 0

    def start_in(g, ib, si):
        gc = jnp.minimum(g, _NCHUNK - 1)
        pltpu.async_copy(src_hbm.at[b, pl.ds(t0 + gc * _CHUNK, _CHUNK), :],
                         ib, si)

    def start_out(g, ob, so):
        pltpu.async_copy(ob,
                         out_hbm.at[b, pl.ds(t0 + g * _CHUNK, _CHUNK), :], so)

    def compute(g, ib, ob):
        def row_body(r, carry):
            ridx = g * _CHUNK + r
            shift_vec = plsc.load_gather(shifts_v, [zero16 + ridx])
            idx0 = (iota - shift_vec) & (_S - 1)
            rvec = zero16 + r
            for j in range(_S // _L):
                elem = (idx0 + (_L * j)) & (_S - 1)
                vec = plsc.load_gather(ib, [rvec, elem])
                ob[r, pl.ds(_L * j, _L)] = vec
            return carry

        lax.fori_loop(0, _CHUNK, row_body, 0)

    start_in(0, inb0, si0)
    start_in(1, inb1, si1)

    def pair_body(k, carry):
        for g_off, (ib, ob, si, so) in enumerate(
            ((inb0, outb0, si0, so0), (inb1, outb1, si1, so1))):
            g = 2 * k + g_off
            pltpu.make_async_copy(src_hbm.at[b, pl.ds(t0, _CHUNK), :], ib, si).wait()

            @pl.when(k > 0)
            def _():
                pltpu.make_async_copy(
                    ob,
                    out_hbm.at[b, pl.ds(t0, _CHUNK), :], so).wait()

            compute(g, ib, ob)
            start_out(g, ob, so)
            start_in(g + 2, ib, si)
        return carry

    lax.fori_loop(0, _NCHUNK // 2, pair_body, 0)

    pltpu.make_async_copy(src_hbm.at[b, pl.ds(t0, _CHUNK), :], inb0, si0).wait()
    pltpu.make_async_copy(src_hbm.at[b, pl.ds(t0, _CHUNK), :], inb1, si1).wait()
    pltpu.make_async_copy(outb0, out_hbm.at[b, pl.ds(t0, _CHUNK), :], so0).wait()
    pltpu.make_async_copy(outb1, out_hbm.at[b, pl.ds(t0, _CHUNK), :], so1).wait()


@jax.jit
def kernel(src, shifts):
    shifts_i32 = shifts.astype(jnp.int32)
    mesh = plsc.VectorSubcoreMesh(core_axis_name="c", subcore_axis_name="s")
    return pl.kernel(
        _roll_body,
        out_type=jax.ShapeDtypeStruct((_B, _T, _S), jnp.float32),
        mesh=mesh,
        compiler_params=pltpu.CompilerParams(
            needs_layout_passes=False, use_tc_tiling_on_sc=False),
        scratch_types=[
            pltpu.VMEM((_ROWS_PER_W,), jnp.int32),
            pltpu.VMEM((_CHUNK, _S), jnp.float32),
            pltpu.VMEM((_CHUNK, _S), jnp.float32),
            pltpu.VMEM((_CHUNK, _S), jnp.float32),
            pltpu.VMEM((_CHUNK, _S), jnp.float32),
            pltpu.SemaphoreType.DMA,
            pltpu.SemaphoreType.DMA,
            pltpu.SemaphoreType.DMA,
            pltpu.SemaphoreType.DMA,
        ],
    )(src, shifts_i32)


# speedup vs baseline: 2.9183x; 2.9183x over previous
import functools

import jax
import jax.numpy as jnp
from jax import lax
from jax.experimental import pallas as pl
from jax.experimental.pallas import tpu as pltpu
from jax.experimental.pallas import tpu_sc as plsc

_B, _T, _S = 16, 2048, 512
_NROWS = _B * _T
_NC, _NS, _L = 2, 16, 16
_NW = _NC * _NS
_ROWS_PER_W = _NROWS // _NW
_CHUNK = 32
_NCHUNK = _ROWS_PER_W // _CHUNK


def _roll_body(src_hbm, shifts_hbm, out_hbm, shifts_v,
               inb0, inb1, outb0, outb1, si0, si1, so0, so1):
    wid = lax.axis_index("s") * _NC + lax.axis_index("c")
    b = wid // 2
    t0 = (wid % 2) * _ROWS_PER_W
    pltpu.sync_copy(shifts_hbm.at[b, pl.ds(t0, _ROWS_PER_W)], shifts_v)

    iota = lax.iota(jnp.int32, _L)
    zero16 = iota * 0

    def start_in(g, ib, si):
        gc = jnp.minimum(g, _NCHUNK - 1)
        pltpu.async_copy(src_hbm.at[b, pl.ds(t0 + gc * _CHUNK, _CHUNK), :],
                         ib, si)

    def start_out(g, ob, so):
        pltpu.async_copy(ob,
                         out_hbm.at[b, pl.ds(t0 + g * _CHUNK, _CHUNK), :], so)

    def compute(g, ib, ob):
        @plsc.parallel_loop(0, _CHUNK, step=1, unroll=2)
        def row_body(r):
            ridx = g * _CHUNK + r
            shift_vec = plsc.load_gather(shifts_v, [zero16 + ridx])
            idx0 = (iota - shift_vec) & (_S - 1)
            rvec = zero16 + r
            for j in range(_S // _L):
                elem = (idx0 + (_L * j)) & (_S - 1)
                vec = plsc.load_gather(ib, [rvec, elem])
                ob[r, pl.ds(_L * j, _L)] = vec

    start_in(0, inb0, si0)
    start_in(1, inb1, si1)

    def pair_body(k, carry):
        for g_off, (ib, ob, si, so) in enumerate(
            ((inb0, outb0, si0, so0), (inb1, outb1, si1, so1))):
            g = 2 * k + g_off
            pltpu.make_async_copy(src_hbm.at[b, pl.ds(t0, _CHUNK), :], ib, si).wait()

            @pl.when(k > 0)
            def _():
                pltpu.make_async_copy(
                    ob,
                    out_hbm.at[b, pl.ds(t0, _CHUNK), :], so).wait()

            compute(g, ib, ob)
            start_out(g, ob, so)
            start_in(g + 2, ib, si)
        return carry

    lax.fori_loop(0, _NCHUNK // 2, pair_body, 0)

    pltpu.make_async_copy(src_hbm.at[b, pl.ds(t0, _CHUNK), :], inb0, si0).wait()
    pltpu.make_async_copy(src_hbm.at[b, pl.ds(t0, _CHUNK), :], inb1, si1).wait()
    pltpu.make_async_copy(outb0, out_hbm.at[b, pl.ds(t0, _CHUNK), :], so0).wait()
    pltpu.make_async_copy(outb1, out_hbm.at[b, pl.ds(t0, _CHUNK), :], so1).wait()


@jax.jit
def kernel(src, shifts):
    shifts_i32 = shifts.astype(jnp.int32)
    mesh = plsc.VectorSubcoreMesh(core_axis_name="c", subcore_axis_name="s")
    return pl.kernel(
        _roll_body,
        out_type=jax.ShapeDtypeStruct((_B, _T, _S), jnp.float32),
        mesh=mesh,
        compiler_params=pltpu.CompilerParams(needs_layout_passes=False),
        scratch_types=[
            pltpu.VMEM((_ROWS_PER_W,), jnp.int32),
            pltpu.VMEM((_CHUNK, _S), jnp.float32),
            pltpu.VMEM((_CHUNK, _S), jnp.float32),
            pltpu.VMEM((_CHUNK, _S), jnp.float32),
            pltpu.VMEM((_CHUNK, _S), jnp.float32),
            pltpu.SemaphoreType.DMA,
            pltpu.SemaphoreType.DMA,
            pltpu.SemaphoreType.DMA,
            pltpu.SemaphoreType.DMA,
        ],
    )(src, shifts_i32)


# speedup vs baseline: 3.2437x; 1.1115x over previous
import functools

import jax
import jax.numpy as jnp
from jax import lax
from jax.experimental import pallas as pl
from jax.experimental.pallas import tpu as pltpu
from jax.experimental.pallas import tpu_sc as plsc

_B, _T, _S = 16, 2048, 512
_NROWS = _B * _T
_NC, _NS, _L = 2, 16, 16
_NW = _NC * _NS
_ROWS_PER_W = _NROWS // _NW
_CHUNK = 32
_NCHUNK = _ROWS_PER_W // _CHUNK


def _roll_body(src_hbm, shifts_hbm, out_hbm, shifts_v,
               inb0, inb1, outb0, outb1, si0, si1, so0, so1):
    wid = lax.axis_index("s") * _NC + lax.axis_index("c")
    b = wid // 2
    t0 = (wid % 2) * _ROWS_PER_W
    pltpu.sync_copy(shifts_hbm.at[b, pl.ds(t0, _ROWS_PER_W)], shifts_v)

    iota = lax.iota(jnp.int32, _L)
    zero16 = iota * 0

    def start_in(g, ib, si):
        gc = jnp.minimum(g, _NCHUNK - 1)
        pltpu.async_copy(src_hbm.at[b, pl.ds(t0 + gc * _CHUNK, _CHUNK), :],
                         ib, si)

    def start_out(g, ob, so):
        pltpu.async_copy(ob,
                         out_hbm.at[b, pl.ds(t0 + g * _CHUNK, _CHUNK), :], so)

    def compute(g, ib, ob):
        @plsc.parallel_loop(0, _CHUNK, step=1, unroll=4)
        def row_body(r):
            ridx = g * _CHUNK + r
            shift_vec = plsc.load_gather(shifts_v, [zero16 + ridx])
            idx0 = (iota - shift_vec) & (_S - 1)
            rvec = zero16 + r
            for j in range(_S // _L):
                elem = (idx0 + (_L * j)) & (_S - 1)
                vec = plsc.load_gather(ib, [rvec, elem])
                ob[r, pl.ds(_L * j, _L)] = vec

    start_in(0, inb0, si0)
    start_in(1, inb1, si1)

    def pair_body(k, carry):
        for g_off, (ib, ob, si, so) in enumerate(
            ((inb0, outb0, si0, so0), (inb1, outb1, si1, so1))):
            g = 2 * k + g_off
            pltpu.make_async_copy(src_hbm.at[b, pl.ds(t0, _CHUNK), :], ib, si).wait()

            @pl.when(k > 0)
            def _():
                pltpu.make_async_copy(
                    ob,
                    out_hbm.at[b, pl.ds(t0, _CHUNK), :], so).wait()

            compute(g, ib, ob)
            start_out(g, ob, so)
            start_in(g + 2, ib, si)
        return carry

    lax.fori_loop(0, _NCHUNK // 2, pair_body, 0)

    pltpu.make_async_copy(src_hbm.at[b, pl.ds(t0, _CHUNK), :], inb0, si0).wait()
    pltpu.make_async_copy(src_hbm.at[b, pl.ds(t0, _CHUNK), :], inb1, si1).wait()
    pltpu.make_async_copy(outb0, out_hbm.at[b, pl.ds(t0, _CHUNK), :], so0).wait()
    pltpu.make_async_copy(outb1, out_hbm.at[b, pl.ds(t0, _CHUNK), :], so1).wait()


@jax.jit
def kernel(src, shifts):
    shifts_i32 = shifts.astype(jnp.int32)
    mesh = plsc.VectorSubcoreMesh(core_axis_name="c", subcore_axis_name="s")
    return pl.kernel(
        _roll_body,
        out_type=jax.ShapeDtypeStruct((_B, _T, _S), jnp.float32),
        mesh=mesh,
        compiler_params=pltpu.CompilerParams(needs_layout_passes=False),
        scratch_types=[
            pltpu.VMEM((_ROWS_PER_W,), jnp.int32),
            pltpu.VMEM((_CHUNK, _S), jnp.float32),
            pltpu.VMEM((_CHUNK, _S), jnp.float32),
            pltpu.VMEM((_CHUNK, _S), jnp.float32),
            pltpu.VMEM((_CHUNK, _S), jnp.float32),
            pltpu.SemaphoreType.DMA,
            pltpu.SemaphoreType.DMA,
            pltpu.SemaphoreType.DMA,
            pltpu.SemaphoreType.DMA,
        ],
    )(src, shifts_i32)
